# baseline (speedup 1.0000x reference)
import functools

import jax
import jax.numpy as jnp
from jax import lax
from jax.experimental import pallas as pl
from jax.experimental.pallas import tpu as pltpu
from jax.experimental.pallas import tpu_sc as plsc

DIM = 1024
HIDDEN = 4096
NUM_EXPERTS = 8
TOP_K = 2
HT = 2048
N_HT = HIDDEN // HT
N_TOK = 64
LANES = 16
N_WORKERS = 32
TOK_PER_W = N_TOK // N_WORKERS


def _lane_rotate(x, shift):
    idx = (lax.broadcasted_iota(jnp.int32, (LANES,), 0) + shift) % LANES
    return x.at[idx].get(mode="promise_in_bounds")


def _all_reduce(x, op):
    for shift in (8, 4, 2, 1):
        x = op(x, _lane_rotate(x, shift))
    return x


def _sc_routing_body(x_hbm, wr_hbm, out_hbm, x_v, wr_v, out_v):
    wid = lax.axis_index("c") * 16 + lax.axis_index("s")
    base = wid * TOK_PER_W
    pltpu.sync_copy(x_hbm.at[pl.ds(base, TOK_PER_W)], x_v)
    pltpu.sync_copy(wr_hbm, wr_v)

    lane = lax.broadcasted_iota(jnp.int32, (LANES,), 0)

    for t in range(TOK_PER_W):
        def chunk(i, accs):
            xc = x_v[t, pl.ds(i * LANES, LANES)]
            return tuple(accs[e] + xc * wr_v[e, pl.ds(i * LANES, LANES)]
                         for e in range(NUM_EXPERTS))

        accs = lax.fori_loop(0, DIM // LANES, chunk,
                             tuple(jnp.zeros((LANES,), jnp.float32)
                                   for _ in range(NUM_EXPERTS)))
        logits = jnp.full((LANES,), -1e30, jnp.float32)
        for e in range(NUM_EXPERTS):
            logits = jnp.where(lane == e, _all_reduce(accs[e], jnp.add), logits)

        m = _all_reduce(logits, jnp.maximum)
        ex = jnp.where(lane < NUM_EXPERTS, jnp.exp(logits - m), 0.0)
        p = ex / _all_reduce(ex, jnp.add)

        m1 = _all_reduce(p, jnp.maximum)
        idx1 = _all_reduce(jnp.where(p == m1, lane, LANES), jnp.minimum)
        mask1 = lane == idx1
        p2 = jnp.where(mask1, -1.0, p)
        m2 = _all_reduce(p2, jnp.maximum)
        idx2 = _all_reduce(jnp.where(p2 == m2, lane, LANES), jnp.minimum)
        mask2 = lane == idx2
        out_v[t, :] = jnp.where(mask1 | mask2, p, 0.0) / (m1 + m2)

    pltpu.sync_copy(out_v, out_hbm.at[pl.ds(base, TOK_PER_W)])


def _sc_routing(x_flat, Wr):
    mesh = plsc.VectorSubcoreMesh(core_axis_name="c", subcore_axis_name="s")
    return pl.kernel(
        _sc_routing_body,
        mesh=mesh,
        out_type=jax.ShapeDtypeStruct((N_TOK, LANES), jnp.float32),
        scratch_types=[
            pltpu.VMEM((TOK_PER_W, DIM), jnp.float32),
            pltpu.VMEM((NUM_EXPERTS, DIM), jnp.float32),
            pltpu.VMEM((TOK_PER_W, LANES), jnp.float32),
        ],
    )(x_flat, Wr)


def _moe_body(x_ref, coeff_ref, w1_ref, w2_ref, out_ref, xt_s, coeff_s, acc_s):
    e = pl.program_id(0)
    h = pl.program_id(1)

    @pl.when((e == 0) & (h == 0))
    def _prologue():
        xt_s[...] = x_ref[...].T
        coeff_s[...] = coeff_ref[...].T

    xt = xt_s[...]
    coeff = coeff_s[...]
    iota_e = jax.lax.broadcasted_iota(jnp.int32, coeff.shape, 0)
    coeff_e = jnp.sum(jnp.where(iota_e == e, coeff, 0.0), axis=0, keepdims=True)

    h1 = jnp.dot(w1_ref[0], xt, preferred_element_type=jnp.float32)
    h1 = h1 * jax.nn.sigmoid(h1)
    part = jnp.dot(w2_ref[0], h1, preferred_element_type=jnp.float32) * coeff_e

    @pl.when((e == 0) & (h == 0))
    def _init():
        acc_s[...] = part

    @pl.when((e > 0) | (h > 0))
    def _acc():
        acc_s[...] += part

    @pl.when((e == NUM_EXPERTS - 1) & (h == N_HT - 1))
    def _epilogue():
        out_ref[...] = acc_s[...].T


@functools.partial(jax.jit, static_argnames=())
def kernel(x, Wr, W1, W2):
    b, s, d = x.shape
    n_tok = b * s
    x_flat = x.reshape(n_tok, d)

    coeff = _sc_routing(x_flat, Wr)

    out = pl.pallas_call(
        _moe_body,
        grid=(NUM_EXPERTS, N_HT),
        in_specs=[
            pl.BlockSpec((n_tok, d), lambda e, h: (0, 0)),
            pl.BlockSpec((n_tok, LANES), lambda e, h: (0, 0)),
            pl.BlockSpec((1, HT, d), lambda e, h: (e, h, 0)),
            pl.BlockSpec((1, d, HT), lambda e, h: (e, 0, h)),
        ],
        out_specs=pl.BlockSpec((n_tok, d), lambda e, h: (0, 0)),
        out_shape=jax.ShapeDtypeStruct((n_tok, d), jnp.float32),
        scratch_shapes=[
            pltpu.VMEM((d, n_tok), jnp.float32),
            pltpu.VMEM((LANES, n_tok), jnp.float32),
            pltpu.VMEM((d, n_tok), jnp.float32),
        ],
        compiler_params=pltpu.CompilerParams(
            dimension_semantics=("arbitrary", "arbitrary"),
        ),
    )(x_flat, coeff, W1, W2)

    return out.reshape(b, s, d)


# speedup vs baseline: 1.2539x; 1.2539x over previous
import functools

import jax
import jax.numpy as jnp
from jax.experimental import pallas as pl
from jax.experimental.pallas import tpu as pltpu

DIM = 1024
HIDDEN = 4096
NUM_EXPERTS = 8
TOP_K = 2
HT = 2048
N_HT = HIDDEN // HT
N_STEPS = NUM_EXPERTS * N_HT
N_TOK = 64
NBUF = 3


def _moe_body(x_ref, wr_ref, w1_hbm, w2_hbm, out_ref,
              xt_s, coeff_s, acc_s, w1_buf, w2_buf, sem1, sem2):

    g = pl.program_id(0)

    def w1_copy(step, buf):
        e, h = step // N_HT, step % N_HT
        return pltpu.make_async_copy(
            w1_hbm.at[e, pl.ds(h * HT, HT), :], w1_buf.at[buf], sem1.at[buf])

    def w2_copy(step, buf):
        e, h = step // N_HT, step % N_HT
        return pltpu.make_async_copy(
            w2_hbm.at[e, :, pl.ds(h * HT, HT)], w2_buf.at[buf], sem2.at[buf])

    @pl.when(g == 0)
    def _prime():
        for s in range(NBUF):
            w1_copy(s, s).start()
            w2_copy(s, s).start()

    @pl.when(g == 0)
    def _prologue():
        xt = x_ref[...].T
        xt_s[...] = xt
        logits = jnp.dot(wr_ref[...], xt, preferred_element_type=jnp.float32)
        m = jnp.max(logits, axis=0, keepdims=True)
        p = jnp.exp(logits - m)
        p = p / jnp.sum(p, axis=0, keepdims=True)

        iota = jax.lax.broadcasted_iota(jnp.int32, p.shape, 0)
        big = jnp.int32(NUM_EXPERTS)
        m1 = jnp.max(p, axis=0, keepdims=True)
        idx1 = jnp.min(jnp.where(p == m1, iota, big), axis=0, keepdims=True)
        mask1 = iota == idx1
        p2 = jnp.where(mask1, -1.0, p)
        m2 = jnp.max(p2, axis=0, keepdims=True)
        idx2 = jnp.min(jnp.where(p2 == m2, iota, big), axis=0, keepdims=True)
        mask2 = iota == idx2
        coeff_s[...] = jnp.where(mask1 | mask2, p, 0.0) / (m1 + m2)

    buf = jax.lax.rem(g, NBUF)
    w1_copy(g, buf).wait()
    w2_copy(g, buf).wait()

    e = g // N_HT
    xt = xt_s[...]
    coeff = coeff_s[...]
    iota_e = jax.lax.broadcasted_iota(jnp.int32, coeff.shape, 0)
    coeff_e = jnp.sum(jnp.where(iota_e == e, coeff, 0.0), axis=0, keepdims=True)

    h1 = jnp.dot(w1_buf[buf], xt, preferred_element_type=jnp.float32)
    h1 = h1 * jax.nn.sigmoid(h1)
    part = jnp.dot(w2_buf[buf], h1, preferred_element_type=jnp.float32) * coeff_e

    @pl.when(g == 0)
    def _init():
        acc_s[...] = part

    @pl.when(g > 0)
    def _acc():
        acc_s[...] += part

    @pl.when(g < N_STEPS - NBUF)
    def _refill():
        w1_copy(g + NBUF, buf).start()
        w2_copy(g + NBUF, buf).start()

    @pl.when(g == N_STEPS - 1)
    def _epilogue():
        out_ref[...] = acc_s[...].T


@functools.partial(jax.jit, static_argnames=())
def kernel(x, Wr, W1, W2):
    b, s, d = x.shape
    n_tok = b * s
    x_flat = x.reshape(n_tok, d)

    out = pl.pallas_call(
        _moe_body,
        grid=(N_STEPS,),
        in_specs=[
            pl.BlockSpec((n_tok, d), lambda g: (0, 0)),
            pl.BlockSpec((NUM_EXPERTS, d), lambda g: (0, 0)),
            pl.BlockSpec(memory_space=pltpu.MemorySpace.HBM),
            pl.BlockSpec(memory_space=pltpu.MemorySpace.HBM),
        ],
        out_specs=pl.BlockSpec((n_tok, d), lambda g: (0, 0)),
        out_shape=jax.ShapeDtypeStruct((n_tok, d), jnp.float32),
        scratch_shapes=[
            pltpu.VMEM((d, n_tok), jnp.float32),
            pltpu.VMEM((NUM_EXPERTS, n_tok), jnp.float32),
            pltpu.VMEM((d, n_tok), jnp.float32),
            pltpu.VMEM((NBUF, HT, DIM), jnp.float32),
            pltpu.VMEM((NBUF, DIM, HT), jnp.float32),
            pltpu.SemaphoreType.DMA((NBUF,)),
            pltpu.SemaphoreType.DMA((NBUF,)),
        ],
        compiler_params=pltpu.CompilerParams(
            dimension_semantics=("arbitrary",),
        ),
    )(x_flat, Wr, W1, W2)

    return out.reshape(b, s, d)
